# baseline (speedup 1.0000x reference)
import functools

import jax
import jax.numpy as jnp
from jax import lax
from jax.experimental import pallas as pl
from jax.experimental.pallas import tpu as pltpu
from jax.experimental.pallas import tpu_sc as plsc

_DIM = 1000
_BATCH = 16384
_LANES = 16
_NW = 32
_COLS_PER_W = _BATCH // _NW
_SLAB = 128
_NSLAB = _COLS_PER_W // _SLAB
_CHUNK_ROWS = 200
_NCHUNK = _DIM // _CHUNK_ROWS


def _one_hot_t(labels):
    mesh = plsc.VectorSubcoreMesh(core_axis_name="c", subcore_axis_name="s")

    @functools.partial(
        pl.kernel,
        mesh=mesh,
        out_type=jax.ShapeDtypeStruct((_DIM, _BATCH), jnp.float32),
        scratch_types=[
            pltpu.VMEM((_COLS_PER_W,), jnp.int32),
            pltpu.VMEM((_DIM, _SLAB), jnp.float32),
            pltpu.SemaphoreType.DMA,
            pltpu.SemaphoreType.DMA,
            pltpu.SemaphoreType.DMA,
            pltpu.SemaphoreType.DMA,
            pltpu.SemaphoreType.DMA,
            pltpu.SemaphoreType.DMA,
        ],
        compiler_params=pltpu.CompilerParams(needs_layout_passes=False),
    )
    def k(labels_hbm, out_hbm, lab_v, buf_v, s0, s1, s2, s3, s4, slab):
        sems = [s0, s1, s2, s3, s4]
        wid = lax.axis_index("c") * (_NW // 2) + lax.axis_index("s")
        col0 = wid * _COLS_PER_W
        lab_handle = pltpu.async_copy(
            labels_hbm.at[pl.ds(col0 * 1, _COLS_PER_W)], lab_v, slab)

        zeros16 = jnp.zeros((_LANES,), jnp.float32)
        ones16 = jnp.ones((_LANES,), jnp.float32)
        iota16 = lax.iota(jnp.int32, _LANES)

        def zero_chunk(kc):
            def zero_body(r, _):
                for cs in range(0, _SLAB, _LANES):
                    buf_v[r, pl.ds(cs, _LANES)] = zeros16
                return 0

            lax.fori_loop(kc * _CHUNK_ROWS, (kc + 1) * _CHUNK_ROWS,
                          zero_body, 0, unroll=4)

        def scatter_masked(s, kc, val):
            r0 = kc * _CHUNK_ROWS
            for g in range(_SLAB // _LANES):
                lab = lab_v[pl.ds(s * _SLAB + g * _LANES, _LANES)]
                mask = (lab >= r0) & (lab < r0 + _CHUNK_ROWS)
                cols = iota16 + (g * _LANES)
                plsc.store_scatter(buf_v, [lab, cols], val, mask=mask)

        def fire(s, kc):
            r0 = kc * _CHUNK_ROWS
            return pltpu.async_copy(
                buf_v.at[pl.ds(r0, _CHUNK_ROWS)],
                out_hbm.at[pl.ds(r0, _CHUNK_ROWS),
                           pl.ds(col0 + s * _SLAB, _SLAB)],
                sems[kc],
            )

        handles = [None] * _NCHUNK
        for kc in range(_NCHUNK):
            zero_chunk(kc)
            if kc == 0:
                lab_handle.wait()
            scatter_masked(0, kc, ones16)
            handles[kc] = fire(0, kc)
        for s in range(1, _NSLAB):
            for kc in range(_NCHUNK):
                handles[kc].wait()
                scatter_masked(s - 1, kc, zeros16)
                scatter_masked(s, kc, ones16)
                handles[kc] = fire(s, kc)
        for kc in range(_NCHUNK):
            handles[kc].wait()

    return k(labels)


def kernel(labels, eye):
    del eye
    return _one_hot_t(labels.astype(jnp.int32)).T


# speedup vs baseline: 1.0662x; 1.0662x over previous
import functools

import jax
import jax.numpy as jnp
from jax import lax
from jax.experimental import pallas as pl
from jax.experimental.pallas import tpu as pltpu
from jax.experimental.pallas import tpu_sc as plsc

_DIM = 1000
_BATCH = 16384
_LANES = 16
_NW = 32
_COLS_PER_W = _BATCH // _NW
_SLAB = 128
_NSLAB = _COLS_PER_W // _SLAB
_CHUNK_ROWS = 200
_NCHUNK = _DIM // _CHUNK_ROWS


def _one_hot_t(labels):
    mesh = plsc.VectorSubcoreMesh(core_axis_name="c", subcore_axis_name="s")

    @functools.partial(
        pl.kernel,
        mesh=mesh,
        out_type=jax.ShapeDtypeStruct((_DIM, _BATCH), jnp.float32),
        scratch_types=[
            pltpu.VMEM((_COLS_PER_W,), jnp.int32),
            pltpu.VMEM((_DIM, _SLAB), jnp.float32),
            pltpu.SemaphoreType.DMA,
            pltpu.SemaphoreType.DMA,
            pltpu.SemaphoreType.DMA,
            pltpu.SemaphoreType.DMA,
            pltpu.SemaphoreType.DMA,
            pltpu.SemaphoreType.DMA,
        ],
        compiler_params=pltpu.CompilerParams(needs_layout_passes=False),
    )
    def k(labels_hbm, out_hbm, lab_v, buf_v, s0, s1, s2, s3, s4, slab):
        sems = [s0, s1, s2, s3, s4]
        wid = lax.axis_index("c") * (_NW // 2) + lax.axis_index("s")
        col0 = wid * _COLS_PER_W
        lab_handle = pltpu.async_copy(
            labels_hbm.at[pl.ds(col0 * 1, _COLS_PER_W)], lab_v, slab)

        zeros16 = jnp.zeros((_LANES,), jnp.float32)
        ones16 = jnp.ones((_LANES,), jnp.float32)
        iota16 = lax.iota(jnp.int32, _LANES)

        def zero_chunk(kc):
            def zero_body(r, _):
                for cs in range(0, _SLAB, _LANES):
                    buf_v[r, pl.ds(cs, _LANES)] = zeros16
                return 0

            lax.fori_loop(kc * _CHUNK_ROWS, (kc + 1) * _CHUNK_ROWS,
                          zero_body, 0)

        def scatter_masked(s, kc, val):
            r0 = kc * _CHUNK_ROWS

            def g_body(g, _):
                lab = lab_v[pl.ds(s * _SLAB + g * _LANES, _LANES)]
                mask = (lab >= r0) & (lab < r0 + _CHUNK_ROWS)
                cols = iota16 + g * _LANES
                plsc.store_scatter(buf_v, [lab, cols], val, mask=mask)
                return 0

            lax.fori_loop(0, _SLAB // _LANES, g_body, 0)

        def fire(s, kc):
            r0 = kc * _CHUNK_ROWS
            return pltpu.async_copy(
                buf_v.at[pl.ds(r0, _CHUNK_ROWS)],
                out_hbm.at[pl.ds(r0, _CHUNK_ROWS),
                           pl.ds(col0 + s * _SLAB, _SLAB)],
                sems[kc],
            )

        handles = [None] * _NCHUNK
        for kc in range(_NCHUNK):
            zero_chunk(kc)
            if kc == 0:
                lab_handle.wait()
            scatter_masked(0, kc, ones16)
            handles[kc] = fire(0, kc)
        for s in range(1, _NSLAB):
            for kc in range(_NCHUNK):
                handles[kc].wait()
                scatter_masked(s - 1, kc, zeros16)
                scatter_masked(s, kc, ones16)
                handles[kc] = fire(s, kc)
        for kc in range(_NCHUNK):
            handles[kc].wait()

    return k(labels)


def kernel(labels, eye):
    del eye
    return _one_hot_t(labels.astype(jnp.int32)).T


# speedup vs baseline: 1.0776x; 1.0107x over previous
import functools

import jax
import jax.numpy as jnp
from jax import lax
from jax.experimental import pallas as pl
from jax.experimental.pallas import tpu as pltpu
from jax.experimental.pallas import tpu_sc as plsc

_DIM = 1000
_BATCH = 16384
_LANES = 16
_NW = 32
_COLS_PER_W = _BATCH // _NW
_SLAB = 128
_NSLAB = _COLS_PER_W // _SLAB
_CHUNK_ROWS = 200
_NCHUNK = _DIM // _CHUNK_ROWS


def _one_hot_t(labels):
    mesh = plsc.VectorSubcoreMesh(core_axis_name="c", subcore_axis_name="s")

    @functools.partial(
        pl.kernel,
        mesh=mesh,
        out_type=jax.ShapeDtypeStruct((_DIM, _BATCH), jnp.float32),
        scratch_types=[
            pltpu.VMEM((_COLS_PER_W,), jnp.int32),
            pltpu.VMEM((_DIM, _SLAB), jnp.float32),
            pltpu.SemaphoreType.DMA,
            pltpu.SemaphoreType.DMA,
            pltpu.SemaphoreType.DMA,
            pltpu.SemaphoreType.DMA,
            pltpu.SemaphoreType.DMA,
            pltpu.SemaphoreType.DMA,
        ],
        compiler_params=pltpu.CompilerParams(needs_layout_passes=False),
    )
    def k(labels_hbm, out_hbm, lab_v, buf_v, s0, s1, s2, s3, s4, slab):
        sems = [s0, s1, s2, s3, s4]
        wid = lax.axis_index("c") * (_NW // 2) + lax.axis_index("s")
        col0 = wid * _COLS_PER_W
        lab_handle = pltpu.async_copy(
            labels_hbm.at[pl.ds(col0 * 1, _COLS_PER_W)], lab_v, slab)

        zeros16 = jnp.zeros((_LANES,), jnp.float32)
        ones16 = jnp.ones((_LANES,), jnp.float32)
        iota16 = lax.iota(jnp.int32, _LANES)

        def zero_chunk(kc):
            def zero_body(r, _):
                for cs in range(0, _SLAB, _LANES):
                    buf_v[r, pl.ds(cs, _LANES)] = zeros16
                return 0

            lax.fori_loop(kc * _CHUNK_ROWS, (kc + 1) * _CHUNK_ROWS,
                          zero_body, 0)

        def scatter_masked(s, kc, val):
            r0 = kc * _CHUNK_ROWS

            def g_body(g, _):
                lab = lab_v[pl.ds(s * _SLAB + g * _LANES, _LANES)]
                mask = (lab >= r0) & (lab < r0 + _CHUNK_ROWS)
                cols = iota16 + g * _LANES
                plsc.store_scatter(buf_v, [lab, cols], val, mask=mask)
                return 0

            lax.fori_loop(0, _SLAB // _LANES, g_body, 0)

        def fire(s, kc):
            r0 = kc * _CHUNK_ROWS
            pltpu.async_copy(
                buf_v.at[pl.ds(r0, _CHUNK_ROWS)],
                out_hbm.at[pl.ds(r0, _CHUNK_ROWS),
                           pl.ds(col0 + s * _SLAB, _SLAB)],
                sems[kc],
            )

        def drain(s, kc):
            r0 = kc * _CHUNK_ROWS
            pltpu.make_async_copy(
                buf_v.at[pl.ds(r0, _CHUNK_ROWS)],
                out_hbm.at[pl.ds(r0, _CHUNK_ROWS),
                           pl.ds(col0 + s * _SLAB, _SLAB)],
                sems[kc],
            ).wait()

        for kc in range(_NCHUNK):
            zero_chunk(kc)
            if kc == 0:
                lab_handle.wait()
            scatter_masked(0, kc, ones16)
            fire(0, kc)

        def s_body(s, _):
            for kc in range(_NCHUNK):
                drain(s - 1, kc)
                scatter_masked(s - 1, kc, zeros16)
                scatter_masked(s, kc, ones16)
                fire(s, kc)
            return 0

        lax.fori_loop(1, _NSLAB, s_body, 0)
        for kc in range(_NCHUNK):
            drain(_NSLAB - 1, kc)

    return k(labels)


def kernel(labels, eye):
    del eye
    return _one_hot_t(labels.astype(jnp.int32)).T


# speedup vs baseline: 1.0918x; 1.0133x over previous
import functools

import jax
import jax.numpy as jnp
from jax import lax
from jax.experimental import pallas as pl
from jax.experimental.pallas import tpu as pltpu
from jax.experimental.pallas import tpu_sc as plsc

_DIM = 1000
_BATCH = 16384
_LANES = 16
_NW = 32
_COLS_PER_W = _BATCH // _NW
_SLAB = 128
_NSLAB = _COLS_PER_W // _SLAB
_CHUNK_ROWS = 200
_NCHUNK = _DIM // _CHUNK_ROWS


def _one_hot_t(labels):
    mesh = plsc.VectorSubcoreMesh(core_axis_name="c", subcore_axis_name="s")

    @functools.partial(
        pl.kernel,
        mesh=mesh,
        out_type=jax.ShapeDtypeStruct((_DIM, _BATCH), jnp.float32),
        scratch_types=[
            pltpu.VMEM((_COLS_PER_W,), jnp.int32),
            pltpu.VMEM((_DIM, _SLAB), jnp.float32),
            pltpu.SemaphoreType.DMA((_NCHUNK,)),
            pltpu.SemaphoreType.DMA,
        ],
        compiler_params=pltpu.CompilerParams(needs_layout_passes=False),
    )
    def k(labels_hbm, out_hbm, lab_v, buf_v, sems, slab):
        wid = lax.axis_index("c") * (_NW // 2) + lax.axis_index("s")
        col0 = wid * _COLS_PER_W
        lab_handle = pltpu.async_copy(
            labels_hbm.at[pl.ds(col0 * 1, _COLS_PER_W)], lab_v, slab)

        zeros16 = jnp.zeros((_LANES,), jnp.float32)
        ones16 = jnp.ones((_LANES,), jnp.float32)
        iota16 = lax.iota(jnp.int32, _LANES)

        def zero_chunk(kc):
            def zero_body(r, _):
                for cs in range(0, _SLAB, _LANES):
                    buf_v[r, pl.ds(cs, _LANES)] = zeros16
                return 0

            lax.fori_loop(kc * _CHUNK_ROWS, (kc + 1) * _CHUNK_ROWS,
                          zero_body, 0)

        def scatter_masked(s, kc, val):
            r0 = kc * _CHUNK_ROWS

            def g_body(g, _):
                lab = lab_v[pl.ds(s * _SLAB + g * _LANES, _LANES)]
                mask = (lab >= r0) & (lab < r0 + _CHUNK_ROWS)
                cols = iota16 + g * _LANES
                plsc.store_scatter(buf_v, [lab, cols], val, mask=mask)
                return 0

            lax.fori_loop(0, _SLAB // _LANES, g_body, 0)

        def fire(s, kc):
            r0 = kc * _CHUNK_ROWS
            pltpu.async_copy(
                buf_v.at[pl.ds(r0, _CHUNK_ROWS)],
                out_hbm.at[pl.ds(r0, _CHUNK_ROWS),
                           pl.ds(col0 + s * _SLAB, _SLAB)],
                sems.at[kc],
            )

        def drain(s, kc):
            r0 = kc * _CHUNK_ROWS
            pltpu.make_async_copy(
                buf_v.at[pl.ds(r0, _CHUNK_ROWS)],
                out_hbm.at[pl.ds(r0, _CHUNK_ROWS),
                           pl.ds(col0 + s * _SLAB, _SLAB)],
                sems.at[kc],
            ).wait()

        def prologue_body(kc, _):
            zero_chunk(kc)
            scatter_masked(0, kc, ones16)
            fire(0, kc)
            return 0

        zero_chunk(0)
        lab_handle.wait()
        scatter_masked(0, 0, ones16)
        fire(0, 0)
        lax.fori_loop(1, _NCHUNK, prologue_body, 0)

        def s_body(s, _):
            def kc_body(kc, _):
                drain(s - 1, kc)
                scatter_masked(s - 1, kc, zeros16)
                scatter_masked(s, kc, ones16)
                fire(s, kc)
                return 0

            lax.fori_loop(0, _NCHUNK, kc_body, 0)
            return 0

        lax.fori_loop(1, _NSLAB, s_body, 0)

        def drain_body(kc, _):
            drain(_NSLAB - 1, kc)
            return 0

        lax.fori_loop(0, _NCHUNK, drain_body, 0)

    return k(labels)


def kernel(labels, eye):
    del eye
    return _one_hot_t(labels.astype(jnp.int32)).T
